# Initial kernel scaffold; baseline (speedup 1.0000x reference)
import functools

import jax
import jax.numpy as jnp
from jax import lax
from jax.experimental import pallas as pl
from jax.experimental.pallas import tpu as pltpu
from jax.experimental.pallas import tpu_sc as plsc

PW = 64


def kernel(images, positions, widths):
    del widths
    n = positions.shape[0]
    info = plsc.get_sparse_core_info()
    nw = info.num_cores * info.num_subcores
    n_per = n // nw
    groups = n_per // 8
    pos_flat = positions.reshape(-1)

    mesh = plsc.VectorSubcoreMesh(core_axis_name="c", subcore_axis_name="s")

    @functools.partial(
        pl.kernel,
        out_type=jax.ShapeDtypeStruct((n, PW, PW), jnp.float32),
        mesh=mesh,
        scratch_types=[
            pltpu.VMEM((2 * n_per,), jnp.int32),
            pltpu.VMEM((16, PW, PW), jnp.float32),
            pltpu.SemaphoreType.DMA,
            pltpu.SemaphoreType.DMA,
        ],
    )
    def patch_kernel(images_h, pos_h, out_h, pos_v, buf_v, in_sem, out_sem):
        wid = lax.axis_index("s") * info.num_cores + lax.axis_index("c")
        base = wid * n_per
        pltpu.sync_copy(pos_h.at[pl.ds(base * 2, 2 * n_per)], pos_v)

        def fire_group(g, slot):
            pv = pos_v[pl.ds(g * 16, 16)]
            copies = []
            for j in range(8):
                r = pv[2 * j]
                c = pv[2 * j + 1]
                copies.append(
                    pltpu.async_copy(
                        images_h.at[pl.ds(r, PW), pl.ds(c, PW)],
                        buf_v.at[slot + j],
                        in_sem,
                    )
                )
            return copies

        def drain_group(g, slot, copies):
            out0 = base + g * 8
            for j in range(8):
                copies[j].wait()
                pltpu.async_copy(
                    buf_v.at[slot + j], out_h.at[out0 + j], out_sem
                ).wait()

        def body(g, carry):
            del carry
            slot = (g % 2) * 8
            copies = fire_group(g, slot)
            drain_group(g, slot, copies)
            return 0

        lax.fori_loop(0, groups, body, 0)

    return patch_kernel(images, pos_flat)


# speedup vs baseline: 38.6547x; 38.6547x over previous
import functools

import jax
import jax.numpy as jnp
from jax import lax
from jax.experimental import pallas as pl
from jax.experimental.pallas import tpu as pltpu
from jax.experimental.pallas import tpu_sc as plsc

PW = 64
BW = PW + 8
NB = 4
PREF = 3


def kernel(images, positions, widths):
    del widths
    n = positions.shape[0]
    info = plsc.get_sparse_core_info()
    nw = info.num_cores * info.num_subcores
    n_per = n // nw
    pos_flat = positions.reshape(-1)

    mesh = plsc.VectorSubcoreMesh(core_axis_name="c", subcore_axis_name="s")

    @functools.partial(
        pl.kernel,
        out_type=jax.ShapeDtypeStruct((n, PW, PW), jnp.float32),
        mesh=mesh,
        compiler_params=pltpu.CompilerParams(
            use_tc_tiling_on_sc=False, needs_layout_passes=False
        ),
        scratch_types=[
            pltpu.VMEM((2 * n_per,), jnp.int32),
            pltpu.VMEM((NB * PW, BW), jnp.float32),
            pltpu.VMEM((2, PW, PW), jnp.float32),
            pltpu.SemaphoreType.DMA,
            pltpu.SemaphoreType.DMA,
        ],
    )
    def patch_kernel(images_h, pos_h, out_h, pos_v, buf_v, obuf_v, in_sem,
                     out_sem):
        wid = lax.axis_index("s") * info.num_cores + lax.axis_index("c")
        base = wid * n_per
        pltpu.sync_copy(pos_h.at[pl.ds(base * 2, 2 * n_per)], pos_v)
        iota = lax.iota(jnp.int32, 16)

        def read_pos(p):
            pv = plsc.load_gather(pos_v, [iota + 2 * p])
            return pv[0], pv[1]

        def fire_in(p):
            r, c = read_pos(p)
            c8 = pl.multiple_of(jnp.bitwise_and(c, -8), 8)
            pltpu.async_copy(
                images_h.at[pl.ds(r, PW), pl.ds(c8, BW)],
                buf_v.at[pl.ds((p % NB) * PW, PW)],
                in_sem,
            )

        for p in range(PREF):
            fire_in(p)

        def body(p, carry):
            del carry
            pf = p + PREF

            @pl.when(pf < n_per)
            def _():
                fire_in(pf)

            pltpu.make_async_copy(
                images_h.at[pl.ds(0, PW), pl.ds(0, BW)],
                buf_v.at[pl.ds((p % NB) * PW, PW)],
                in_sem,
            ).wait()

            @pl.when(p >= 2)
            def _():
                pltpu.make_async_copy(
                    obuf_v.at[0], out_h.at[0], out_sem
                ).wait()

            _, c = read_pos(p)
            svec = jnp.bitwise_and(jnp.full((16,), c, jnp.int32), 7)
            slot_row = (p % NB) * PW
            ob = p % 2
            col_idx = [svec + iota + 16 * k for k in range(4)]
            for row in range(PW):
                row_vec = jnp.full((16,), slot_row + row, jnp.int32)
                for k in range(4):
                    v = plsc.load_gather(buf_v, [row_vec, col_idx[k]])
                    obuf_v[ob, row, pl.ds(16 * k, 16)] = v

            pltpu.async_copy(obuf_v.at[ob], out_h.at[base + p], out_sem)
            return 0

        lax.fori_loop(0, n_per, body, 0)

        for _ in range(2):
            pltpu.make_async_copy(obuf_v.at[0], out_h.at[0], out_sem).wait()

    return patch_kernel(images, pos_flat)
